# baseline (speedup 1.0000x reference)
import functools
import math

import jax
import jax.numpy as jnp
from jax import lax
from jax.experimental import pallas as pl
from jax.experimental.pallas import tpu as pltpu, tpu_sc as plsc

_B, _H, _W, _C, _N = 16, 32, 32, 80, 32
_T = _B * _N
_G = _B * _H * _W
_HW = _H * _W
_PI2 = math.pi ** 2


def _vperm(v, idx):
    return jnp.take_along_axis(v, idx, axis=0, mode="promise_in_bounds")


def _sc_dedup(tbT, labs):
    info = plsc.get_sparse_core_info()
    nc, ns, L = info.num_cores, info.num_subcores, info.num_lanes
    nw = nc * ns
    tpw = _T // nw
    mesh = plsc.VectorSubcoreMesh(core_axis_name="c", subcore_axis_name="s")

    @functools.partial(
        pl.kernel,
        mesh=mesh,
        compiler_params=pltpu.CompilerParams(use_tc_tiling_on_sc=False),
        out_type=jax.ShapeDtypeStruct((_B, 8, 128), jnp.float32),
        scratch_types=[
            pltpu.VMEM((tpw,), jnp.float32),
            pltpu.VMEM((tpw,), jnp.float32),
            pltpu.VMEM((tpw,), jnp.float32),
            pltpu.VMEM((tpw,), jnp.float32),
            pltpu.VMEM((tpw,), jnp.float32),
            pltpu.VMEM((tpw,), jnp.float32),
            pltpu.VMEM((tpw,), jnp.int32),
            pltpu.VMEM((tpw,), jnp.int32),
            pltpu.VMEM((8 * tpw,), jnp.float32),
            pltpu.SemaphoreType.DMA,
            pltpu.SemaphoreType.DMA,
        ],
    )
    def sc(tbT_hbm, labs_hbm, out_hbm,
           xa_v, xb_v, ya_v, yb_v, w_v, h_v, la_v, lb_v, pk_v, sin, sout):
        wid = lax.axis_index("s") * nc + lax.axis_index("c")
        base = wid * tpw
        img = wid // 2
        par = wid - img * 2
        ibase = img * _N
        copies = [
            pltpu.async_copy(tbT_hbm.at[0, pl.ds(ibase, tpw)], xa_v, sin),
            pltpu.async_copy(tbT_hbm.at[0, pl.ds(ibase + tpw, tpw)], xb_v,
                             sin),
            pltpu.async_copy(tbT_hbm.at[1, pl.ds(ibase, tpw)], ya_v, sin),
            pltpu.async_copy(tbT_hbm.at[1, pl.ds(ibase + tpw, tpw)], yb_v,
                             sin),
            pltpu.async_copy(tbT_hbm.at[2, pl.ds(base, tpw)], w_v, sin),
            pltpu.async_copy(tbT_hbm.at[3, pl.ds(base, tpw)], h_v, sin),
            pltpu.async_copy(labs_hbm.at[pl.ds(ibase, tpw)], la_v, sin),
            pltpu.async_copy(labs_hbm.at[pl.ds(ibase + tpw, tpw)], lb_v,
                             sin),
        ]
        for cp in copies:
            cp.wait()

        def cell_of(x, y):
            gx = jnp.minimum(jnp.maximum(x * float(_W), 0.0),
                             float(_W - 1)).astype(jnp.int32)
            gy = jnp.minimum(jnp.maximum(y * float(_H), 0.0),
                             float(_H - 1)).astype(jnp.int32)
            return gy * _W + gx + img * _HW

        xa, xb, ya, yb = xa_v[...], xb_v[...], ya_v[...], yb_v[...]
        la, lb = la_v[...], lb_v[...]
        cell_a = cell_of(xa, ya)
        cell_b = cell_of(xb, yb)
        key_a = (cell_a << 7) | la
        key_b = (cell_b << 7) | lb
        second = par == 1
        cell = jnp.where(second, cell_b, cell_a)
        key = jnp.where(second, key_b, key_a)
        labf = jnp.where(second, lb, la).astype(jnp.float32)
        x = jnp.where(second, xb, xa)
        y = jnp.where(second, yb, ya)

        lanes = lax.iota(jnp.int32, L)
        one = jnp.full((L,), 1, jnp.int32)
        zero = jnp.full((L,), 0, jnp.int32)
        firstm = jnp.where(par == 0, one, zero)
        loser_c = zero
        loser_k = zero
        for k in range(1, tpw):
            idx = (lanes + k) & (L - 1)
            pk = _vperm(key, idx)
            m = jnp.where(lanes < (tpw - k), one, zero)
            loser_c = loser_c | (jnp.where((pk >> 7) == cell, one, zero) & m)
            loser_k = loser_k | (jnp.where(pk == key, one, zero) & m)
        for k in range(tpw):
            idx = (lanes + k) & (L - 1)
            pk = _vperm(key_b, idx)
            loser_c = loser_c | (jnp.where((pk >> 7) == cell, one, zero)
                                 & firstm)
            loser_k = loser_k | (jnp.where(pk == key, one, zero) & firstm)

        winner = jnp.where(loser_c == 0, 1.0, 0.0)
        winner2 = jnp.where(loser_k == 0, 1.0, 0.0)
        fields = (cell.astype(jnp.float32), winner, winner2, labf,
                  x, y, w_v[...], h_v[...])
        for k, val in enumerate(fields):
            pk_v[pl.ds(k * tpw, tpw)] = val
        outs = [
            pltpu.async_copy(pk_v.at[pl.ds(k * tpw, tpw)],
                             out_hbm.at[img, k, pl.ds(par * tpw, tpw)], sout)
            for k in range(8)
        ]
        for cp in outs:
            cp.wait()

    return sc(tbT, labs)


def _softplus(x):
    return jnp.maximum(x, 0.0) + jnp.log1p(jnp.exp(-jnp.abs(x)))


def _atan_pos(x):
    big = x > 2.414213562373095
    mid = x > 0.4142135623730950
    xr = jnp.where(big, -1.0 / (x + 1e-30),
                   jnp.where(mid, (x - 1.0) / (x + 1.0), x))
    bias = jnp.where(big, math.pi / 2, jnp.where(mid, math.pi / 4, 0.0))
    z = xr * xr
    y = (((8.05374449538e-2 * z - 1.38776856032e-1) * z
          + 1.99777106478e-1) * z - 3.33329491539e-1) * z * xr + xr
    return bias + y


def _tdot(a, b):
    return lax.dot_general(a, b, (((0,), (0,)), ((), ())),
                           preferred_element_type=jnp.float32)


def _tc_body(bbox_ref, obj_ref, cls_ref, pk_ref,
             tot_ref, objl_ref, boxl_ref, clsl_ref, acc):
    eps = 1e-7
    b = pl.program_id(0)
    f32 = jnp.float32
    i32 = jnp.int32
    nt = 2 * _N
    ncell = 2 * _HW
    pk = jnp.concatenate([pk_ref[0][:, 0:_N], pk_ref[1][:, 0:_N]],
                         axis=1)
    cellf = pk[0:1, :]
    win = pk[1:2, :]
    w2 = pk[2:3, :]
    lab = pk[3:4, :].astype(i32)
    tx, ty = pk[4:5, :], pk[5:6, :]
    tw, th = pk[6:7, :], pk[7:8, :]
    cells_loc = cellf.astype(i32) - b * ncell

    pcell = lax.broadcasted_iota(i32, (ncell, nt), 0)
    PT = (pcell == cells_loc).astype(f32)
    bb4 = jnp.concatenate([bbox_ref[0], bbox_ref[1]], axis=1)
    bbg5 = lax.dot_general(bb4, PT, (((1,), (0,)), ((), ())),
                           preferred_element_type=f32)
    obf = obj_ref[0]
    objg = (lax.dot_general(obf[0:1, :], PT[0:_HW, :],
                            (((1,), (0,)), ((), ())),
                            preferred_element_type=f32)
            + lax.dot_general(obf[1:2, :], PT[_HW:, :],
                              (((1,), (0,)), ((), ())),
                              preferred_element_type=f32))
    clsgT = _tdot(cls_ref[...], PT)

    npos_p = jnp.sum(win)
    s_og = _softplus(objg)
    a1_p = jnp.sum(win * (s_og - objg))
    a2_p = jnp.sum(win * s_og)
    sall_p = jnp.sum(_softplus(obf))

    sig = 1.0 / (1.0 + jnp.exp(-bbg5))
    px, py = sig[0:1, :], sig[1:2, :]
    pw, ph = sig[2:3, :], sig[3:4, :]
    x11, y11 = px - pw / 2, py - ph / 2
    x12, y12 = px + pw / 2, py + ph / 2
    x21, y21 = tx - tw / 2, ty - th / 2
    x22, y22 = tx + tw / 2, ty + th / 2
    w1, h1 = x12 - x11, y12 - y11
    w2_, h2_ = x22 - x21, y22 - y21
    inter_w = jnp.maximum(jnp.minimum(x12, x22) - jnp.maximum(x11, x21), 0.0)
    inter_h = jnp.maximum(jnp.minimum(y12, y22) - jnp.maximum(y11, y21), 0.0)
    inter = inter_w * inter_h
    union = w1 * h1 + w2_ * h2_ - inter + eps
    iou = inter / union
    cw = jnp.maximum(x12, x22) - jnp.minimum(x11, x21)
    ch = jnp.maximum(y12, y22) - jnp.minimum(y11, y21)
    c2 = cw * cw + ch * ch + eps
    rho2 = ((x21 + x22 - x11 - x12) ** 2 + (y21 + y22 - y11 - y12) ** 2) / 4.0
    v = (4.0 / _PI2) * (_atan_pos(w2_ / (h2_ + eps))
                        - _atan_pos(w1 / (h1 + eps))) ** 2
    alpha = v / (1.0 - iou + v + eps)
    ciou = iou - (rho2 / c2 + alpha * v)
    box_p = jnp.sum(win * (1.0 - ciou))

    rs = jnp.sum(_softplus(clsgT), axis=0, keepdims=True)
    onehotT = (lax.broadcasted_iota(i32, (_C, 2 * _N), 0) == lab)
    picked = jnp.sum(clsgT * onehotT.astype(f32), axis=0, keepdims=True)
    cls1_p = jnp.sum(win * rs)
    cls2_p = jnp.sum(w2 * picked)

    parts = (npos_p, a1_p, a2_p, sall_p, box_p, cls1_p, cls2_p)
    for i, p in enumerate(parts):
        acc[i] = jnp.where(b == 0, p, acc[i] + p)

    @pl.when(b == pl.num_programs(0) - 1)
    def _():
        n_pos = jnp.maximum(acc[0], 1.0)
        n_neg = jnp.maximum(float(_G) - acc[0], 1.0)
        obj_loss = acc[1] / n_pos + 0.5 * (acc[3] - acc[2]) / n_neg
        box_loss = acc[4] / n_pos
        cls_loss = (acc[5] - acc[6]) / (n_pos * float(_C))
        total = obj_loss + 5.0 * box_loss + 1.0 * cls_loss
        tot_ref[...] = jnp.reshape(total, (1, 1))
        objl_ref[...] = jnp.reshape(obj_loss, (1, 1))
        boxl_ref[...] = jnp.reshape(box_loss, (1, 1))
        clsl_ref[...] = jnp.reshape(cls_loss, (1, 1))


def _tc_loss(bbox4, objf, cls2d, packed):
    out = jax.ShapeDtypeStruct((1, 1), jnp.float32)
    o_spec = pl.BlockSpec((1, 1), lambda b: (0, 0))
    return pl.pallas_call(
        _tc_body,
        grid=(_B // 2,),
        in_specs=[
            pl.BlockSpec((2, 4, _HW), lambda b: (b, 0, 0)),
            pl.BlockSpec((1, 2, _HW), lambda b: (b, 0, 0)),
            pl.BlockSpec((2 * _HW, _C), lambda b: (b, 0)),
            pl.BlockSpec((2, 8, 128), lambda b: (b, 0, 0)),
        ],
        out_specs=(o_spec, o_spec, o_spec, o_spec),
        out_shape=(out, out, out, out),
        scratch_shapes=[pltpu.SMEM((8,), jnp.float32)],
    )(bbox4, objf, cls2d, packed)


def kernel(obj, bbox, cls, target_boxes, target_labels):
    cls2d = cls.reshape(_G, _C)
    bbox4 = bbox.transpose(0, 3, 1, 2).reshape(_B, 4, _HW)
    objf = obj.reshape(_B // 2, 2, _HW)
    tbT = jnp.transpose(target_boxes, (2, 0, 1)).reshape(4, _T)
    labs = target_labels.reshape(_T).astype(jnp.int32)
    packed = _sc_dedup(tbT, labs)
    tot, objl, boxl, clsl = _tc_loss(bbox4, objf, cls2d, packed)
    return (tot[0, 0], objl[0, 0], boxl[0, 0], clsl[0, 0])


# speedup vs baseline: 1.0235x; 1.0235x over previous
import functools
import math

import jax
import jax.numpy as jnp
from jax import lax
from jax.experimental import pallas as pl
from jax.experimental.pallas import tpu as pltpu, tpu_sc as plsc

_B, _H, _W, _C, _N = 16, 32, 32, 80, 32
_T = _B * _N
_G = _B * _H * _W
_HW = _H * _W
_PI2 = math.pi ** 2


def _vperm(v, idx):
    return jnp.take_along_axis(v, idx, axis=0, mode="promise_in_bounds")


def _sc_dedup(tbT, labs):
    info = plsc.get_sparse_core_info()
    nc, ns, L = info.num_cores, info.num_subcores, info.num_lanes
    nw = nc * ns
    tpw = _T // nw
    mesh = plsc.VectorSubcoreMesh(core_axis_name="c", subcore_axis_name="s")

    @functools.partial(
        pl.kernel,
        mesh=mesh,
        compiler_params=pltpu.CompilerParams(use_tc_tiling_on_sc=False),
        out_type=jax.ShapeDtypeStruct((_B, 8, 128), jnp.float32),
        scratch_types=[
            pltpu.VMEM((tpw,), jnp.float32),
            pltpu.VMEM((tpw,), jnp.float32),
            pltpu.VMEM((tpw,), jnp.float32),
            pltpu.VMEM((tpw,), jnp.float32),
            pltpu.VMEM((tpw,), jnp.float32),
            pltpu.VMEM((tpw,), jnp.float32),
            pltpu.VMEM((tpw,), jnp.int32),
            pltpu.VMEM((tpw,), jnp.int32),
            pltpu.VMEM((8 * tpw,), jnp.float32),
            pltpu.SemaphoreType.DMA,
            pltpu.SemaphoreType.DMA,
        ],
    )
    def sc(tbT_hbm, labs_hbm, out_hbm,
           xa_v, xb_v, ya_v, yb_v, w_v, h_v, la_v, lb_v, pk_v, sin, sout):
        wid = lax.axis_index("s") * nc + lax.axis_index("c")
        base = wid * tpw
        img = wid // 2
        par = wid - img * 2
        ibase = img * _N
        copies = [
            pltpu.async_copy(tbT_hbm.at[0, pl.ds(ibase, tpw)], xa_v, sin),
            pltpu.async_copy(tbT_hbm.at[0, pl.ds(ibase + tpw, tpw)], xb_v,
                             sin),
            pltpu.async_copy(tbT_hbm.at[1, pl.ds(ibase, tpw)], ya_v, sin),
            pltpu.async_copy(tbT_hbm.at[1, pl.ds(ibase + tpw, tpw)], yb_v,
                             sin),
            pltpu.async_copy(tbT_hbm.at[2, pl.ds(base, tpw)], w_v, sin),
            pltpu.async_copy(tbT_hbm.at[3, pl.ds(base, tpw)], h_v, sin),
            pltpu.async_copy(labs_hbm.at[pl.ds(ibase, tpw)], la_v, sin),
            pltpu.async_copy(labs_hbm.at[pl.ds(ibase + tpw, tpw)], lb_v,
                             sin),
        ]
        for cp in copies:
            cp.wait()

        def cell_of(x, y):
            gx = jnp.minimum(jnp.maximum(x * float(_W), 0.0),
                             float(_W - 1)).astype(jnp.int32)
            gy = jnp.minimum(jnp.maximum(y * float(_H), 0.0),
                             float(_H - 1)).astype(jnp.int32)
            return gy * _W + gx + img * _HW

        xa, xb, ya, yb = xa_v[...], xb_v[...], ya_v[...], yb_v[...]
        la, lb = la_v[...], lb_v[...]
        cell_a = cell_of(xa, ya)
        cell_b = cell_of(xb, yb)
        key_a = (cell_a << 7) | la
        key_b = (cell_b << 7) | lb
        second = par == 1
        cell = jnp.where(second, cell_b, cell_a)
        key = jnp.where(second, key_b, key_a)
        labf = jnp.where(second, lb, la).astype(jnp.float32)
        x = jnp.where(second, xb, xa)
        y = jnp.where(second, yb, ya)

        lanes = lax.iota(jnp.int32, L)
        one = jnp.full((L,), 1, jnp.int32)
        zero = jnp.full((L,), 0, jnp.int32)
        firstm = jnp.where(par == 0, one, zero)
        loser_c = zero
        loser_k = zero
        for k in range(1, tpw):
            idx = (lanes + k) & (L - 1)
            pk = _vperm(key, idx)
            m = jnp.where(lanes < (tpw - k), one, zero)
            loser_c = loser_c | (jnp.where((pk >> 7) == cell, one, zero) & m)
            loser_k = loser_k | (jnp.where(pk == key, one, zero) & m)
        for k in range(tpw):
            idx = (lanes + k) & (L - 1)
            pk = _vperm(key_b, idx)
            loser_c = loser_c | (jnp.where((pk >> 7) == cell, one, zero)
                                 & firstm)
            loser_k = loser_k | (jnp.where(pk == key, one, zero) & firstm)

        winner = jnp.where(loser_c == 0, 1.0, 0.0)
        winner2 = jnp.where(loser_k == 0, 1.0, 0.0)
        fields = (cell.astype(jnp.float32), winner, winner2, labf,
                  x, y, w_v[...], h_v[...])
        for k, val in enumerate(fields):
            pk_v[pl.ds(k * tpw, tpw)] = val
        outs = [
            pltpu.async_copy(pk_v.at[pl.ds(k * tpw, tpw)],
                             out_hbm.at[img, k, pl.ds(par * tpw, tpw)], sout)
            for k in range(8)
        ]
        for cp in outs:
            cp.wait()

    return sc(tbT, labs)


def _softplus(x):
    return jnp.maximum(x, 0.0) + jnp.log1p(jnp.exp(-jnp.abs(x)))


def _atan_pos(x):
    big = x > 2.414213562373095
    mid = x > 0.4142135623730950
    xr = jnp.where(big, -1.0 / (x + 1e-30),
                   jnp.where(mid, (x - 1.0) / (x + 1.0), x))
    bias = jnp.where(big, math.pi / 2, jnp.where(mid, math.pi / 4, 0.0))
    z = xr * xr
    y = (((8.05374449538e-2 * z - 1.38776856032e-1) * z
          + 1.99777106478e-1) * z - 3.33329491539e-1) * z * xr + xr
    return bias + y


def _tdot(a, b):
    return lax.dot_general(a, b, (((0,), (0,)), ((), ())),
                           preferred_element_type=jnp.float32)


def _tc_body(bbox_ref, cls_ref, pk_ref,
             tot_ref, objl_ref, boxl_ref, clsl_ref, acc):
    eps = 1e-7
    b = pl.program_id(0)
    f32 = jnp.float32
    i32 = jnp.int32
    nt = 2 * _N
    ncell = 2 * _HW
    pk = jnp.concatenate([pk_ref[0][:, 0:_N], pk_ref[1][:, 0:_N]],
                         axis=1)
    cellf = pk[0:1, :]
    win = pk[1:2, :]
    w2 = pk[2:3, :]
    lab = pk[3:4, :].astype(i32)
    tx, ty = pk[4:5, :], pk[5:6, :]
    tw, th = pk[6:7, :], pk[7:8, :]
    cells_loc = cellf.astype(i32) - b * ncell

    pcell = lax.broadcasted_iota(i32, (ncell, nt), 0)
    PT = (pcell == cells_loc).astype(f32)
    bb5 = jnp.concatenate([bbox_ref[0], bbox_ref[1]], axis=1)
    bbg5 = lax.dot_general(bb5, PT, (((1,), (0,)), ((), ())),
                           preferred_element_type=f32)
    objg = bbg5[4:5, :]
    clsgT = _tdot(cls_ref[...], PT)

    npos_p = jnp.sum(win)
    s_og = _softplus(objg)
    a1_p = jnp.sum(win * (s_og - objg))
    a2_p = jnp.sum(win * s_og)
    sall_p = jnp.sum(_softplus(bb5[4:5, :]))

    sig = 1.0 / (1.0 + jnp.exp(-bbg5[0:4, :]))
    px, py = sig[0:1, :], sig[1:2, :]
    pw, ph = sig[2:3, :], sig[3:4, :]
    x11, y11 = px - pw / 2, py - ph / 2
    x12, y12 = px + pw / 2, py + ph / 2
    x21, y21 = tx - tw / 2, ty - th / 2
    x22, y22 = tx + tw / 2, ty + th / 2
    w1, h1 = x12 - x11, y12 - y11
    w2_, h2_ = x22 - x21, y22 - y21
    inter_w = jnp.maximum(jnp.minimum(x12, x22) - jnp.maximum(x11, x21), 0.0)
    inter_h = jnp.maximum(jnp.minimum(y12, y22) - jnp.maximum(y11, y21), 0.0)
    inter = inter_w * inter_h
    union = w1 * h1 + w2_ * h2_ - inter + eps
    iou = inter / union
    cw = jnp.maximum(x12, x22) - jnp.minimum(x11, x21)
    ch = jnp.maximum(y12, y22) - jnp.minimum(y11, y21)
    c2 = cw * cw + ch * ch + eps
    rho2 = ((x21 + x22 - x11 - x12) ** 2 + (y21 + y22 - y11 - y12) ** 2) / 4.0
    v = (4.0 / _PI2) * (_atan_pos(w2_ / (h2_ + eps))
                        - _atan_pos(w1 / (h1 + eps))) ** 2
    alpha = v / (1.0 - iou + v + eps)
    ciou = iou - (rho2 / c2 + alpha * v)
    box_p = jnp.sum(win * (1.0 - ciou))

    rs = jnp.sum(_softplus(clsgT), axis=0, keepdims=True)
    onehotT = (lax.broadcasted_iota(i32, (_C, 2 * _N), 0) == lab)
    picked = jnp.sum(clsgT * onehotT.astype(f32), axis=0, keepdims=True)
    cls1_p = jnp.sum(win * rs)
    cls2_p = jnp.sum(w2 * picked)

    parts = (npos_p, a1_p, a2_p, sall_p, box_p, cls1_p, cls2_p)
    for i, p in enumerate(parts):
        acc[i] = jnp.where(b == 0, p, acc[i] + p)

    @pl.when(b == pl.num_programs(0) - 1)
    def _():
        n_pos = jnp.maximum(acc[0], 1.0)
        n_neg = jnp.maximum(float(_G) - acc[0], 1.0)
        obj_loss = acc[1] / n_pos + 0.5 * (acc[3] - acc[2]) / n_neg
        box_loss = acc[4] / n_pos
        cls_loss = (acc[5] - acc[6]) / (n_pos * float(_C))
        total = obj_loss + 5.0 * box_loss + 1.0 * cls_loss
        tot_ref[...] = jnp.reshape(total, (1, 1))
        objl_ref[...] = jnp.reshape(obj_loss, (1, 1))
        boxl_ref[...] = jnp.reshape(box_loss, (1, 1))
        clsl_ref[...] = jnp.reshape(cls_loss, (1, 1))


def _tc_loss(bbox5, cls2d, packed):
    out = jax.ShapeDtypeStruct((1, 1), jnp.float32)
    o_spec = pl.BlockSpec((1, 1), lambda b: (0, 0))
    return pl.pallas_call(
        _tc_body,
        grid=(_B // 2,),
        in_specs=[
            pl.BlockSpec((2, 5, _HW), lambda b: (b, 0, 0)),
            pl.BlockSpec((2 * _HW, _C), lambda b: (b, 0)),
            pl.BlockSpec((2, 8, 128), lambda b: (b, 0, 0)),
        ],
        out_specs=(o_spec, o_spec, o_spec, o_spec),
        out_shape=(out, out, out, out),
        scratch_shapes=[pltpu.SMEM((8,), jnp.float32)],
    )(bbox5, cls2d, packed)


def kernel(obj, bbox, cls, target_boxes, target_labels):
    cls2d = cls.reshape(_G, _C)
    bbox5 = jnp.concatenate(
        [bbox.transpose(0, 3, 1, 2).reshape(_B, 4, _HW),
         obj.reshape(_B, 1, _HW)], axis=1)
    tbT = jnp.transpose(target_boxes, (2, 0, 1)).reshape(4, _T)
    labs = target_labels.reshape(_T).astype(jnp.int32)
    packed = _sc_dedup(tbT, labs)
    tot, objl, boxl, clsl = _tc_loss(bbox5, cls2d, packed)
    return (tot[0, 0], objl[0, 0], boxl[0, 0], clsl[0, 0])
